# baseline (speedup 1.0000x reference)
import jax
import jax.numpy as jnp
from jax.experimental import pallas as pl
from jax.experimental.pallas import tpu as pltpu

_BATCH, _STEPS, _NODES = 64, 24, 64
_D, _QDIM, _PLM = 32, 5, 768
_ALPHA = 0.5
_BB = 8
_XF = _D + 1
_R = _BB * _NODES
_PREC = jax.lax.Precision.DEFAULT


def _tile5(a, qvb):
    return jnp.concatenate([a] * _QDIM, axis=1) * qvb


def _dots8(M, X, w):
    return jnp.concatenate(
        [jax.lax.dot(M[b], X[b * _NODES:(b + 1) * _NODES, :w],
                     precision=_PREC) for b in range(_BB)], axis=0)


def _rnn_body(obs_ref, mask_ref, maskT_ref, avg_ref, avgT_ref, len_ref,
              vpr_ref, rW_ref, Wf1_ref, bf1_ref, Wf2_ref, bf2_ref,
              Wg1_ref, bg1_ref, Wg2_ref, bg2_ref,
              Wxru_ref, Whru_ref, Wxc_ref, Whc_ref, bru_ref, bc_ref,
              out_ref,
              adjE_s, adjW_s, qvx_s, qvh_s, bbru_s, bbc_s, mcol_s, rcol_s):

    @pl.when(pl.program_id(0) == 0)
    def _prologue():
        vpr = vpr_ref[...]
        qh = jnp.maximum(
            jax.lax.dot(vpr, Wf1_ref[...], precision=_PREC) + bf1_ref[...], 0.0)
        qv = jax.lax.dot(qh, Wf2_ref[...], precision=_PREC) + bf2_ref[...]
        gh = jnp.maximum(
            jax.lax.dot(vpr, Wg1_ref[...], precision=_PREC) + bg1_ref[...], 0.0)
        ne = jax.lax.dot(gh, Wg2_ref[...], precision=_PREC) + bg2_ref[...]
        nrm = jnp.sqrt(jnp.sum(ne * ne, axis=1, keepdims=True))
        ne = ne / jnp.maximum(nrm, 1e-12)
        logits = jax.lax.dot_general(ne, ne, (((1,), (1,)), ((), ())),
                                     precision=_PREC)
        mx = jnp.max(logits, axis=1, keepdims=True)
        e = jnp.exp(logits - mx)
        adj = e / jnp.sum(e, axis=1, keepdims=True)
        eye = (jax.lax.broadcasted_iota(jnp.int32, (_NODES, _NODES), 0) ==
               jax.lax.broadcasted_iota(jnp.int32, (_NODES, _NODES), 1)
               ).astype(jnp.float32)
        adjE = adj * (1.0 - eye)
        adjE_s[...] = adjE
        adjW_s[...] = adjE * rW_ref[...]
        qx = jnp.concatenate(
            [jnp.broadcast_to(qv[:, d:d + 1], (_NODES, _XF))
             for d in range(_QDIM)], axis=1)
        qvx_s[...] = jnp.concatenate([qx] * _BB, axis=0)
        qh_ = jnp.concatenate(
            [jnp.broadcast_to(qv[:, d:d + 1], (_NODES, _D))
             for d in range(_QDIM)], axis=1)
        qvh_s[...] = jnp.concatenate([qh_] * _BB, axis=0)
        bbru = jax.lax.dot(qv, bru_ref[...], precision=_PREC)
        bbru_s[...] = jnp.concatenate([bbru] * _BB, axis=0)
        bbc = jax.lax.dot(qv, bc_ref[...], precision=_PREC)
        bbc_s[...] = jnp.concatenate([bbc] * _BB, axis=0)

    vto = jnp.sum(mask_ref[...], axis=1)
    mT_all = maskT_ref[0]
    aT_all = avgT_ref[0]
    vtoT = jnp.sum(mT_all, axis=0)
    rarT_all = _ALPHA * jnp.tanh(aT_all / (vtoT[None] + 1.0))
    mcol_s[...] = jnp.concatenate(
        [mT_all[:, :, b:b + 1] for b in range(_BB)], axis=1)
    rcol_s[...] = jnp.concatenate(
        [rarT_all[:, :, b:b + 1] for b in range(_BB)], axis=1)
    lb = len_ref[...]
    ls = jnp.concatenate(
        [jnp.broadcast_to(lb[b:b + 1, :], (_NODES, 1)) for b in range(_BB)],
        axis=0)
    adjE = adjE_s[...]
    adjW = adjW_s[...]
    qvx = qvx_s[...]
    qvh = qvh_s[...]
    bbru = bbru_s[...]
    bbc = bbc_s[...]
    Wxru = Wxru_ref[...]
    Whru = Whru_ref[...]
    Wxc = Wxc_ref[...]
    Whc = Whc_ref[...]

    def step_fn(step, carry):
        h, out = carry
        m_col = mcol_s[step]
        rar_col = rcol_s[step]
        rar_lane = _ALPHA * jnp.tanh(avg_ref[:, step, :] / (vto + 1.0))
        dr = jnp.abs(rar_col.reshape(_BB, _NODES, 1)
                     - rar_lane[:, None, :])
        Mm = adjE[None] - adjW[None] * dr
        obs = obs_ref[:, step].reshape(_R, _D)
        xr = jnp.concatenate([obs, rar_col], axis=1)
        comb_x = m_col * _dots8(Mm, m_col * xr, _XF) + xr
        t_ru_x = jax.lax.dot(_tile5(comb_x, qvx), Wxru, precision=_PREC)
        t_c_x = jax.lax.dot(_tile5(xr, qvx), Wxc, precision=_PREC)
        ch = m_col * _dots8(Mm, m_col * h, _D) + h
        acc = (jax.lax.dot(_tile5(ch, qvh), Whru, precision=_PREC)
               + t_ru_x + bbru)
        r = jax.nn.sigmoid(acc[:, :_D])
        u = jax.nn.sigmoid(acc[:, _D:2 * _D])
        mgt = m_col > 0.0
        h_r = jnp.where(mgt, r * h, h)
        cand = jnp.tanh(jax.lax.dot(_tile5(h_r, qvh), Whc, precision=_PREC)
                        + t_c_x + bbc)
        h_new = jnp.where(mgt, (1.0 - u) * h_r + u * cand, h)
        out_new = jnp.where(ls == step + 1, h_new, out)
        return h_new, out_new

    h0 = jnp.zeros((_R, _D), jnp.float32)
    _, out = jax.lax.fori_loop(0, _STEPS, step_fn, (h0, h0))
    out_ref[...] = out.reshape(_BB, _NODES, _D)


def kernel(obs_emb, observed_mask, lengths, avg_interval, var_plm_rep,
           rarity_W, Wf1, bf1, Wf2, bf2, Wg1, bg1, Wg2, bg2,
           Wu, bu, Wr, br, Wc, bc):
    maskT = (observed_mask.transpose(1, 2, 0)
             .reshape(_STEPS, _NODES, _BATCH // _BB, _BB)
             .transpose(2, 0, 1, 3))
    avgT = (avg_interval.transpose(1, 2, 0)
            .reshape(_STEPS, _NODES, _BATCH // _BB, _BB)
            .transpose(2, 0, 1, 3))
    Wxru = jnp.stack([Wr[:, :_XF], Wu[:, :_XF]], axis=2).reshape(
        _QDIM * _XF, 2 * _D)
    Whru = jnp.stack([Wr[:, _XF:], Wu[:, _XF:]], axis=2).reshape(
        _QDIM * _D, 2 * _D)
    Wxc = Wc[:, :_XF].reshape(_QDIM * _XF, _D)
    Whc = Wc[:, _XF:].reshape(_QDIM * _D, _D)
    bru = jnp.concatenate([br, bu], axis=1)

    full = lambda nd: (lambda i: (0,) * nd)
    out = pl.pallas_call(
        _rnn_body,
        grid=(_BATCH // _BB,),
        in_specs=[
            pl.BlockSpec((_BB, _STEPS, _NODES, _D), lambda i: (i, 0, 0, 0)),
            pl.BlockSpec((_BB, _STEPS, _NODES), lambda i: (i, 0, 0)),
            pl.BlockSpec((1, _STEPS, _NODES, _BB), lambda i: (i, 0, 0, 0)),
            pl.BlockSpec((_BB, _STEPS, _NODES), lambda i: (i, 0, 0)),
            pl.BlockSpec((1, _STEPS, _NODES, _BB), lambda i: (i, 0, 0, 0)),
            pl.BlockSpec((_BB, 1), lambda i: (i, 0)),
            pl.BlockSpec((_NODES, _PLM), full(2)),
            pl.BlockSpec((_NODES, _NODES), full(2)),
            pl.BlockSpec((_PLM, 2 * _D), full(2)),
            pl.BlockSpec((1, 2 * _D), full(2)),
            pl.BlockSpec((2 * _D, _QDIM), full(2)),
            pl.BlockSpec((1, _QDIM), full(2)),
            pl.BlockSpec((_PLM, 2 * _D), full(2)),
            pl.BlockSpec((1, 2 * _D), full(2)),
            pl.BlockSpec((2 * _D, 8), full(2)),
            pl.BlockSpec((1, 8), full(2)),
            pl.BlockSpec((_QDIM * _XF, 2 * _D), full(2)),
            pl.BlockSpec((_QDIM * _D, 2 * _D), full(2)),
            pl.BlockSpec((_QDIM * _XF, _D), full(2)),
            pl.BlockSpec((_QDIM * _D, _D), full(2)),
            pl.BlockSpec((_QDIM, 2 * _D), full(2)),
            pl.BlockSpec((_QDIM, _D), full(2)),
        ],
        out_specs=pl.BlockSpec((_BB, _NODES, _D), lambda i: (i, 0, 0)),
        out_shape=jax.ShapeDtypeStruct((_BATCH, _NODES, _D), jnp.float32),
        scratch_shapes=[
            pltpu.VMEM((_NODES, _NODES), jnp.float32),
            pltpu.VMEM((_NODES, _NODES), jnp.float32),
            pltpu.VMEM((_R, _QDIM * _XF), jnp.float32),
            pltpu.VMEM((_R, _QDIM * _D), jnp.float32),
            pltpu.VMEM((_R, 2 * _D), jnp.float32),
            pltpu.VMEM((_R, _D), jnp.float32),
            pltpu.VMEM((_STEPS, _R, 1), jnp.float32),
            pltpu.VMEM((_STEPS, _R, 1), jnp.float32),
        ],
        compiler_params=pltpu.CompilerParams(
            dimension_semantics=("arbitrary",)),
    )(obs_emb, observed_mask, maskT, avg_interval, avgT, lengths,
      var_plm_rep, rarity_W, Wf1, bf1.reshape(1, -1), Wf2, bf2.reshape(1, -1),
      Wg1, bg1.reshape(1, -1), Wg2, bg2.reshape(1, -1),
      Wxru, Whru, Wxc, Whc, bru, bc)
    return out


# speedup vs baseline: 1.2127x; 1.2127x over previous
import jax
import jax.numpy as jnp
from jax.experimental import pallas as pl
from jax.experimental.pallas import tpu as pltpu

_BATCH, _STEPS, _NODES = 64, 24, 64
_D, _QDIM, _PLM = 32, 5, 768
_ALPHA = 0.5
_BB = 8
_XF = _D + 1
_NF = 2 * _D + 1
_R = _BB * _NODES
_PREC = jax.lax.Precision.DEFAULT


def _tile5(a, qvb):
    return jnp.concatenate([a] * _QDIM, axis=1) * qvb


def _dots8(M, X, w):
    return jnp.concatenate(
        [jax.lax.dot(M[b], X[b * _NODES:(b + 1) * _NODES, :w],
                     precision=_PREC) for b in range(_BB)], axis=0)


def _rnn_body(obs_ref, mask_ref, maskT_ref, avg_ref, avgT_ref, len_ref,
              vpr_ref, rW_ref, Wf1_ref, bf1_ref, Wf2_ref, bf2_ref,
              Wg1_ref, bg1_ref, Wg2_ref, bg2_ref,
              Wru_ref, Wcf_ref, bru_ref, bc_ref,
              out_ref,
              adjE_s, adjW_s, qvx_s, bbru_s, bbc_s, mcol_s, rcol_s):

    @pl.when(pl.program_id(0) == 0)
    def _prologue():
        vpr = vpr_ref[...]
        qh = jnp.maximum(
            jax.lax.dot(vpr, Wf1_ref[...], precision=_PREC) + bf1_ref[...], 0.0)
        qv = jax.lax.dot(qh, Wf2_ref[...], precision=_PREC) + bf2_ref[...]
        gh = jnp.maximum(
            jax.lax.dot(vpr, Wg1_ref[...], precision=_PREC) + bg1_ref[...], 0.0)
        ne = jax.lax.dot(gh, Wg2_ref[...], precision=_PREC) + bg2_ref[...]
        nrm = jnp.sqrt(jnp.sum(ne * ne, axis=1, keepdims=True))
        ne = ne / jnp.maximum(nrm, 1e-12)
        logits = jax.lax.dot_general(ne, ne, (((1,), (1,)), ((), ())),
                                     precision=_PREC)
        mx = jnp.max(logits, axis=1, keepdims=True)
        e = jnp.exp(logits - mx)
        adj = e / jnp.sum(e, axis=1, keepdims=True)
        eye = (jax.lax.broadcasted_iota(jnp.int32, (_NODES, _NODES), 0) ==
               jax.lax.broadcasted_iota(jnp.int32, (_NODES, _NODES), 1)
               ).astype(jnp.float32)
        adjE = adj * (1.0 - eye)
        adjE_s[...] = adjE
        adjW_s[...] = adjE * rW_ref[...]
        qx = jnp.concatenate(
            [jnp.broadcast_to(qv[:, d:d + 1], (_NODES, _NF))
             for d in range(_QDIM)], axis=1)
        qvx_s[...] = jnp.concatenate([qx] * _BB, axis=0)
        bbru = jax.lax.dot(qv, bru_ref[...], precision=_PREC)
        bbru_s[...] = jnp.concatenate([bbru] * _BB, axis=0)
        bbc = jax.lax.dot(qv, bc_ref[...], precision=_PREC)
        bbc_s[...] = jnp.concatenate([bbc] * _BB, axis=0)

    vto = jnp.sum(mask_ref[...], axis=1)
    mT_all = maskT_ref[0]
    aT_all = avgT_ref[0]
    vtoT = jnp.sum(mT_all, axis=0)
    rarT_all = _ALPHA * jnp.tanh(aT_all / (vtoT[None] + 1.0))
    mcol_s[...] = jnp.concatenate(
        [mT_all[:, :, b:b + 1] for b in range(_BB)], axis=1)
    rcol_s[...] = jnp.concatenate(
        [rarT_all[:, :, b:b + 1] for b in range(_BB)], axis=1)
    lb = len_ref[...]
    ls = jnp.concatenate(
        [jnp.broadcast_to(lb[b:b + 1, :], (_NODES, 1)) for b in range(_BB)],
        axis=0)
    adjE = adjE_s[...]
    adjW = adjW_s[...]
    qvx = qvx_s[...]
    bbru = bbru_s[...]
    bbc = bbc_s[...]
    Wru = Wru_ref[...]
    Wcf = Wcf_ref[...]

    def step_fn(step, carry):
        h, out = carry
        m_col = mcol_s[step]
        rar_col = rcol_s[step]
        rar_lane = _ALPHA * jnp.tanh(avg_ref[:, step, :] / (vto + 1.0))
        dr = jnp.abs(rar_col.reshape(_BB, _NODES, 1)
                     - rar_lane[:, None, :])
        Mm = adjE[None] - adjW[None] * dr
        obs = obs_ref[:, step].reshape(_R, _D)
        xh = jnp.concatenate([obs, rar_col, h], axis=1)
        comb = m_col * _dots8(Mm, m_col * xh, _NF) + xh
        acc = (jax.lax.dot(_tile5(comb, qvx), Wru, precision=_PREC)
               + bbru)
        r = jax.nn.sigmoid(acc[:, :_D])
        u = jax.nn.sigmoid(acc[:, _D:2 * _D])
        mgt = m_col > 0.0
        h_r = jnp.where(mgt, r * h, h)
        xc = jnp.concatenate([obs, rar_col, h_r], axis=1)
        cand = jnp.tanh(jax.lax.dot(_tile5(xc, qvx), Wcf, precision=_PREC)
                        + bbc)
        h_new = jnp.where(mgt, (1.0 - u) * h_r + u * cand, h)
        out_new = jnp.where(ls == step + 1, h_new, out)
        return h_new, out_new

    h0 = jnp.zeros((_R, _D), jnp.float32)
    _, out = jax.lax.fori_loop(0, _STEPS, step_fn, (h0, h0))
    out_ref[...] = out.reshape(_BB, _NODES, _D)


def kernel(obs_emb, observed_mask, lengths, avg_interval, var_plm_rep,
           rarity_W, Wf1, bf1, Wf2, bf2, Wg1, bg1, Wg2, bg2,
           Wu, bu, Wr, br, Wc, bc):
    maskT = (observed_mask.transpose(1, 2, 0)
             .reshape(_STEPS, _NODES, _BATCH // _BB, _BB)
             .transpose(2, 0, 1, 3))
    avgT = (avg_interval.transpose(1, 2, 0)
            .reshape(_STEPS, _NODES, _BATCH // _BB, _BB)
            .transpose(2, 0, 1, 3))
    Wru = jnp.stack([Wr, Wu], axis=2).reshape(_QDIM * _NF, 2 * _D)
    Wcf = Wc.reshape(_QDIM * _NF, _D)
    bru = jnp.concatenate([br, bu], axis=1)

    full = lambda nd: (lambda i: (0,) * nd)
    out = pl.pallas_call(
        _rnn_body,
        grid=(_BATCH // _BB,),
        in_specs=[
            pl.BlockSpec((_BB, _STEPS, _NODES, _D), lambda i: (i, 0, 0, 0)),
            pl.BlockSpec((_BB, _STEPS, _NODES), lambda i: (i, 0, 0)),
            pl.BlockSpec((1, _STEPS, _NODES, _BB), lambda i: (i, 0, 0, 0)),
            pl.BlockSpec((_BB, _STEPS, _NODES), lambda i: (i, 0, 0)),
            pl.BlockSpec((1, _STEPS, _NODES, _BB), lambda i: (i, 0, 0, 0)),
            pl.BlockSpec((_BB, 1), lambda i: (i, 0)),
            pl.BlockSpec((_NODES, _PLM), full(2)),
            pl.BlockSpec((_NODES, _NODES), full(2)),
            pl.BlockSpec((_PLM, 2 * _D), full(2)),
            pl.BlockSpec((1, 2 * _D), full(2)),
            pl.BlockSpec((2 * _D, _QDIM), full(2)),
            pl.BlockSpec((1, _QDIM), full(2)),
            pl.BlockSpec((_PLM, 2 * _D), full(2)),
            pl.BlockSpec((1, 2 * _D), full(2)),
            pl.BlockSpec((2 * _D, 8), full(2)),
            pl.BlockSpec((1, 8), full(2)),
            pl.BlockSpec((_QDIM * _NF, 2 * _D), full(2)),
            pl.BlockSpec((_QDIM * _NF, _D), full(2)),
            pl.BlockSpec((_QDIM, 2 * _D), full(2)),
            pl.BlockSpec((_QDIM, _D), full(2)),
        ],
        out_specs=pl.BlockSpec((_BB, _NODES, _D), lambda i: (i, 0, 0)),
        out_shape=jax.ShapeDtypeStruct((_BATCH, _NODES, _D), jnp.float32),
        scratch_shapes=[
            pltpu.VMEM((_NODES, _NODES), jnp.float32),
            pltpu.VMEM((_NODES, _NODES), jnp.float32),
            pltpu.VMEM((_R, _QDIM * _NF), jnp.float32),
            pltpu.VMEM((_R, 2 * _D), jnp.float32),
            pltpu.VMEM((_R, _D), jnp.float32),
            pltpu.VMEM((_STEPS, _R, 1), jnp.float32),
            pltpu.VMEM((_STEPS, _R, 1), jnp.float32),
        ],
        compiler_params=pltpu.CompilerParams(
            dimension_semantics=("arbitrary",)),
    )(obs_emb, observed_mask, maskT, avg_interval, avgT, lengths,
      var_plm_rep, rarity_W, Wf1, bf1.reshape(1, -1), Wf2, bf2.reshape(1, -1),
      Wg1, bg1.reshape(1, -1), Wg2, bg2.reshape(1, -1),
      Wru, Wcf, bru, bc)
    return out


# speedup vs baseline: 2.4004x; 1.9793x over previous
import jax
import jax.numpy as jnp
from jax.experimental import pallas as pl
from jax.experimental.pallas import tpu as pltpu

_BATCH, _STEPS, _NODES = 64, 24, 64
_D, _QDIM, _PLM = 32, 5, 768
_ALPHA = 0.5
_BB = 8
_NF = 2 * _D + 1
_H2 = 2 * _D
_PREC = jax.lax.Precision.DEFAULT


def _rnn_body(obsT_ref, mask_ref, maskT_ref, avg_ref, avgT_ref, len_ref,
              vprT_ref, rWT_ref, Wf1T_ref, bf1_ref, Wf2T_ref, bf2_ref,
              Wg1T_ref, bg1_ref, Wg2T_ref, bg2_ref,
              WruT_ref, WcT_ref, bruT_ref, bcT_ref,
              out_ref,
              adjET_s, adjWT_s, qv5_s, bbru_s, bbc_s, rrow_s):

    @pl.when(pl.program_id(0) == 0)
    def _prologue():
        vprT = vprT_ref[...]
        qhT = jnp.maximum(
            jax.lax.dot(Wf1T_ref[...], vprT, precision=_PREC) + bf1_ref[...],
            0.0)
        qvT = jax.lax.dot(Wf2T_ref[...], qhT, precision=_PREC) + bf2_ref[...]
        ghT = jnp.maximum(
            jax.lax.dot(Wg1T_ref[...], vprT, precision=_PREC) + bg1_ref[...],
            0.0)
        neT = jax.lax.dot(Wg2T_ref[...], ghT, precision=_PREC) + bg2_ref[...]
        nrm = jnp.sqrt(jnp.sum(neT * neT, axis=0, keepdims=True))
        neT = neT / jnp.maximum(nrm, 1e-12)
        logits = jax.lax.dot_general(neT, neT, (((0,), (0,)), ((), ())),
                                     precision=_PREC)
        mx = jnp.max(logits, axis=0, keepdims=True)
        e = jnp.exp(logits - mx)
        adjT = e / jnp.sum(e, axis=0, keepdims=True)
        eye = (jax.lax.broadcasted_iota(jnp.int32, (_NODES, _NODES), 0) ==
               jax.lax.broadcasted_iota(jnp.int32, (_NODES, _NODES), 1)
               ).astype(jnp.float32)
        adjET = adjT * (1.0 - eye)
        adjET_s[...] = adjET
        adjWT_s[...] = adjET * rWT_ref[...]
        qv5_s[...] = jnp.concatenate(
            [jnp.broadcast_to(qvT[d:d + 1, :], (_NF, _NODES))
             for d in range(_QDIM)], axis=0)
        bbru_s[...] = jax.lax.dot(bruT_ref[...], qvT, precision=_PREC)
        bbc_s[...] = jax.lax.dot(bcT_ref[...], qvT, precision=_PREC)

    vto = jnp.sum(mask_ref[...], axis=1)
    vtoT = jnp.sum(maskT_ref[0], axis=0)
    rrow_s[...] = _ALPHA * jnp.tanh(avgT_ref[0] / (vtoT[None] + 1.0))
    lb3 = len_ref[...].reshape(_BB, 1, 1)
    adjET = adjET_s[...]
    adjWT = adjWT_s[...]
    qv5 = qv5_s[...]
    bbru = bbru_s[...]
    bbc = bbc_s[...]
    WruT = WruT_ref[...]
    WcT = WcT_ref[...]

    def step_fn(step, carry):
        hT, outT = carry
        m3 = mask_ref[:, step, :][:, None, :]
        rar = _ALPHA * jnp.tanh(avg_ref[:, step, :] / (vto + 1.0))
        rar3 = rar[:, None, :]
        rT = rrow_s[step]
        rar_rows = jnp.stack([rT[:, b:b + 1] for b in range(_BB)], axis=0)
        drT = jnp.abs(rar_rows - rar3)
        MmT = adjET[None] - adjWT[None] * drT
        obsT = obsT_ref[:, step]
        xhT = jnp.concatenate([obsT, rar3, hT], axis=1)
        xhmT = m3 * xhT
        combT = m3 * jnp.stack(
            [jax.lax.dot(xhmT[b], MmT[b], precision=_PREC)
             for b in range(_BB)], axis=0) + xhT
        accT = jnp.stack(
            [jax.lax.dot(
                WruT,
                jnp.concatenate([combT[b]] * _QDIM, axis=0) * qv5,
                precision=_PREC) for b in range(_BB)], axis=0) + bbru[None]
        r = jax.nn.sigmoid(accT[:, :_D])
        u = jax.nn.sigmoid(accT[:, _D:_H2])
        mgt = m3 > 0.0
        h_rT = jnp.where(mgt, r * hT, hT)
        xcT = jnp.concatenate([obsT, rar3, h_rT], axis=1)
        candT = jnp.tanh(jnp.stack(
            [jax.lax.dot(
                WcT,
                jnp.concatenate([xcT[b]] * _QDIM, axis=0) * qv5,
                precision=_PREC) for b in range(_BB)], axis=0) + bbc[None])
        h_new = jnp.where(mgt, (1.0 - u) * h_rT + u * candT, hT)
        out_new = jnp.where(lb3 == step + 1, h_new, outT)
        return h_new, out_new

    h0 = jnp.zeros((_BB, _D, _NODES), jnp.float32)
    _, outT = jax.lax.fori_loop(0, _STEPS, step_fn, (h0, h0))
    out_ref[...] = outT


def kernel(obs_emb, observed_mask, lengths, avg_interval, var_plm_rep,
           rarity_W, Wf1, bf1, Wf2, bf2, Wg1, bg1, Wg2, bg2,
           Wu, bu, Wr, br, Wc, bc):
    obsT = obs_emb.transpose(0, 1, 3, 2)
    maskT = (observed_mask.transpose(1, 2, 0)
             .reshape(_STEPS, _NODES, _BATCH // _BB, _BB)
             .transpose(2, 0, 1, 3))
    avgT = (avg_interval.transpose(1, 2, 0)
            .reshape(_STEPS, _NODES, _BATCH // _BB, _BB)
            .transpose(2, 0, 1, 3))
    WruT = jnp.stack([Wr, Wu], axis=2).reshape(
        _QDIM * _NF, 2 * _D).T
    WcT = Wc.reshape(_QDIM * _NF, _D).T
    bruT = jnp.concatenate([br, bu], axis=1).T
    bcT = bc.T

    full = lambda nd: (lambda i: (0,) * nd)
    outT = pl.pallas_call(
        _rnn_body,
        grid=(_BATCH // _BB,),
        in_specs=[
            pl.BlockSpec((_BB, _STEPS, _D, _NODES), lambda i: (i, 0, 0, 0)),
            pl.BlockSpec((_BB, _STEPS, _NODES), lambda i: (i, 0, 0)),
            pl.BlockSpec((1, _STEPS, _NODES, _BB), lambda i: (i, 0, 0, 0)),
            pl.BlockSpec((_BB, _STEPS, _NODES), lambda i: (i, 0, 0)),
            pl.BlockSpec((1, _STEPS, _NODES, _BB), lambda i: (i, 0, 0, 0)),
            pl.BlockSpec((_BB, 1), lambda i: (i, 0)),
            pl.BlockSpec((_PLM, _NODES), full(2)),
            pl.BlockSpec((_NODES, _NODES), full(2)),
            pl.BlockSpec((_H2, _PLM), full(2)),
            pl.BlockSpec((_H2, 1), full(2)),
            pl.BlockSpec((_QDIM, _H2), full(2)),
            pl.BlockSpec((_QDIM, 1), full(2)),
            pl.BlockSpec((_H2, _PLM), full(2)),
            pl.BlockSpec((_H2, 1), full(2)),
            pl.BlockSpec((8, _H2), full(2)),
            pl.BlockSpec((8, 1), full(2)),
            pl.BlockSpec((2 * _D, _QDIM * _NF), full(2)),
            pl.BlockSpec((_D, _QDIM * _NF), full(2)),
            pl.BlockSpec((2 * _D, _QDIM), full(2)),
            pl.BlockSpec((_D, _QDIM), full(2)),
        ],
        out_specs=pl.BlockSpec((_BB, _D, _NODES), lambda i: (i, 0, 0)),
        out_shape=jax.ShapeDtypeStruct((_BATCH, _D, _NODES), jnp.float32),
        scratch_shapes=[
            pltpu.VMEM((_NODES, _NODES), jnp.float32),
            pltpu.VMEM((_NODES, _NODES), jnp.float32),
            pltpu.VMEM((_QDIM * _NF, _NODES), jnp.float32),
            pltpu.VMEM((2 * _D, _NODES), jnp.float32),
            pltpu.VMEM((_D, _NODES), jnp.float32),
            pltpu.VMEM((_STEPS, _NODES, _BB), jnp.float32),
        ],
        compiler_params=pltpu.CompilerParams(
            dimension_semantics=("arbitrary",)),
    )(obsT, observed_mask, maskT, avg_interval, avgT, lengths,
      var_plm_rep.T, rarity_W.T, Wf1.T, bf1.reshape(-1, 1),
      Wf2.T, bf2.reshape(-1, 1), Wg1.T, bg1.reshape(-1, 1),
      Wg2.T, bg2.reshape(-1, 1), WruT, WcT, bruT, bcT)
    return outT.transpose(0, 2, 1)
